# baseline (speedup 1.0000x reference)
import functools

import jax
import jax.numpy as jnp
from jax import lax
from jax.experimental import pallas as pl
from jax.experimental.pallas import tpu as pltpu
from jax.experimental.pallas import tpu_sc as plsc

_NC = 2
_NS = 16
_NW = _NC * _NS
_LANES = 16


@functools.lru_cache(maxsize=None)
def _make_sc_gather(V, D, TOT):
    CHUNK = 128
    per_w = TOT // _NW
    n_chunks = per_w // CHUNK
    scale = float(D) ** 0.5
    mesh = plsc.VectorSubcoreMesh(core_axis_name="c", subcore_axis_name="s")

    @functools.partial(
        pl.kernel,
        mesh=mesh,
        out_type=jax.ShapeDtypeStruct((TOT, D), jnp.float32),
        scratch_types=[
            pltpu.VMEM((n_chunks, CHUNK), jnp.int32),
            pltpu.VMEM((CHUNK, D), jnp.float32),
            pltpu.SemaphoreType.DMA,
        ],
        compiler_params=pltpu.CompilerParams(use_tc_tiling_on_sc=False),
    )
    def sc_kernel(x_hbm, table_hbm, out_hbm, idx_v, rows_v, sem):
        wid = lax.axis_index("s") * _NC + lax.axis_index("c")
        base = wid * per_w
        pltpu.sync_copy(x_hbm.at[wid], idx_v)

        @pl.loop(0, n_chunks)
        def _chunk(j):
            pltpu.async_copy(table_hbm.at[idx_v.at[j]], rows_v, sem).wait()

            @pl.loop(0, CHUNK)
            def _row(r):
                for c in range(D // _LANES):
                    sl = pl.ds(c * _LANES, _LANES)
                    rows_v[r, sl] = rows_v[r, sl] * scale

            pltpu.sync_copy(rows_v, out_hbm.at[pl.ds(base + j * CHUNK, CHUNK)])

    return sc_kernel


def kernel(x, table):
    B, H = x.shape
    V, D = table.shape
    TOT = B * H
    sc = _make_sc_gather(V, D, TOT)
    xr = x.reshape(_NW, TOT // _NW // 128, 128).astype(jnp.int32)
    out = sc(xr, table)
    return out.reshape(B, H, D)


# speedup vs baseline: 1.2098x; 1.2098x over previous
import functools

import jax
import jax.numpy as jnp
from jax import lax
from jax.experimental import pallas as pl
from jax.experimental.pallas import tpu as pltpu
from jax.experimental.pallas import tpu_sc as plsc

_NC = 2
_NS = 16
_NW = _NC * _NS
_LANES = 16
_CHUNK = 128
_NBUF = 5


@functools.lru_cache(maxsize=None)
def _make_sc_gather(V, D, TOT):
    per_w = TOT // _NW
    n_chunks = per_w // _CHUNK
    assert n_chunks % _NBUF == 0
    scale = float(D) ** 0.5
    mesh = plsc.VectorSubcoreMesh(core_axis_name="c", subcore_axis_name="s")

    @functools.partial(
        pl.kernel,
        mesh=mesh,
        out_type=jax.ShapeDtypeStruct((TOT, D), jnp.float32),
        scratch_types=[
            pltpu.VMEM((n_chunks, _CHUNK), jnp.int32),
            pltpu.VMEM((_NBUF, _CHUNK, D), jnp.float32),
            pltpu.SemaphoreType.DMA((_NBUF,)),
            pltpu.SemaphoreType.DMA((_NBUF,)),
        ],
        compiler_params=pltpu.CompilerParams(use_tc_tiling_on_sc=False),
    )
    def sc_kernel(x_hbm, table_hbm, out_hbm, idx_v, rows_v, gsem, wsem):
        wid = lax.axis_index("s") * _NC + lax.axis_index("c")
        base = wid * per_w
        pltpu.sync_copy(x_hbm.at[wid], idx_v)

        def gather_start(j, b):
            pltpu.async_copy(table_hbm.at[idx_v.at[j]], rows_v.at[b], gsem.at[b])

        def gather_wait(j, b):
            pltpu.make_async_copy(
                table_hbm.at[idx_v.at[j]], rows_v.at[b], gsem.at[b]).wait()

        def write_start(j, b):
            pltpu.async_copy(
                rows_v.at[b], out_hbm.at[pl.ds(base + j * _CHUNK, _CHUNK)],
                wsem.at[b])

        def write_wait(b):
            pltpu.make_async_copy(
                rows_v.at[b], out_hbm.at[pl.ds(base, _CHUNK)], wsem.at[b]).wait()

        for b in range(_NBUF - 1):
            gather_start(b, b)

        @pl.loop(0, n_chunks, step=_NBUF)
        def _group(j0):
            for b in range(_NBUF):
                j = j0 + b
                b2 = (b + _NBUF - 1) % _NBUF
                jn = j + _NBUF - 1

                @pl.when(jn < n_chunks)
                def _():
                    @pl.when(j > 0)
                    def _():
                        write_wait(b2)
                    gather_start(jn, b2)

                gather_wait(j, b)

                @plsc.parallel_loop(0, _CHUNK, unroll=4)
                def _row(r):
                    for c in range(D // _LANES):
                        sl = pl.ds(c * _LANES, _LANES)
                        rows_v[b, r, sl] = rows_v[b, r, sl] * scale

                write_start(j, b)

        for b in range(_NBUF):
            write_wait(b)

    return sc_kernel


def kernel(x, table):
    B, H = x.shape
    V, D = table.shape
    TOT = B * H
    sc = _make_sc_gather(V, D, TOT)
    xr = x.reshape(_NW, TOT // _NW // _CHUNK, _CHUNK).astype(jnp.int32)
    out = sc(xr, table)
    return out.reshape(B, H, D)
